# baseline (speedup 1.0000x reference)
import functools

import jax
import jax.numpy as jnp
from jax import lax
from jax.experimental import pallas as pl
from jax.experimental.pallas import tpu as pltpu
from jax.experimental.pallas import tpu_sc as plsc

N_NODES = 50000
HIDDEN = 256
NUM_GRAPHS = 64
NW = 32
SC_END = 18432
R_PER_W = 576
CHUNK = 128
NCHUNKS = 5
PATCH_PER_W = 1024
NSUPER_W = 16
PIDS_LEN = 8 + PATCH_PER_W + 96
LANES = 16
NVJ = HIDDEN // LANES
IDS_PAD = 8
NEG_INF = float("-inf")


def _sc_segment_max(z, batch_i32):
  mesh = plsc.VectorSubcoreMesh(core_axis_name="c", subcore_axis_name="s")

  @functools.partial(
      pl.kernel,
      mesh=mesh,
      compiler_params=pltpu.CompilerParams(needs_layout_passes=False),
      out_type=jax.ShapeDtypeStruct((NW, NUM_GRAPHS, HIDDEN), jnp.float32),
      scratch_types=[
          pltpu.VMEM((CHUNK, HIDDEN), jnp.float32),
          pltpu.VMEM((CHUNK, HIDDEN), jnp.float32),
          pltpu.VMEM((IDS_PAD + R_PER_W + 24,), jnp.int32),
          pltpu.VMEM((NUM_GRAPHS, HIDDEN), jnp.float32),
          pltpu.VMEM((NUM_GRAPHS + LANES,), jnp.int32),
          pltpu.VMEM((NUM_GRAPHS + LANES,), jnp.int32),
          pltpu.VMEM((PIDS_LEN,), jnp.int32),
          pltpu.SemaphoreType.DMA,
          pltpu.SemaphoreType.DMA,
          pltpu.SemaphoreType.DMA,
          pltpu.SemaphoreType.DMA,
      ],
  )
  def body(z_hbm, batch_hbm, out_hbm, buf0, buf1, ids, acc, bstart, bend,
           pids, sem0, sem1, sem_ids, sem_pids):
    def sget(ref, idx):
      return ref[pl.ds(idx, LANES)][0]

    wid = lax.axis_index("c") * jnp.int32(16) + lax.axis_index("s")
    rbase = wid * R_PER_W
    rcount = jnp.minimum(R_PER_W, SC_END - rbase)
    ids_base = jnp.minimum(rbase, SC_END - R_PER_W)
    off0 = rbase - ids_base

    ids_h = pltpu.async_copy(batch_hbm.at[pl.ds(ids_base, R_PER_W)],
                             ids.at[pl.ds(IDS_PAD, R_PER_W)], sem_ids)
    pbase = SC_END + wid * PATCH_PER_W
    pids_base = jnp.minimum(pbase, N_NODES - PATCH_PER_W)
    poff = pbase - pids_base
    pids_h = pltpu.async_copy(batch_hbm.at[pl.ds(pids_base, PATCH_PER_W)],
                              pids.at[pl.ds(IDS_PAD, PATCH_PER_W)], sem_pids)
    neg = jnp.full((LANES,), NEG_INF, dtype=jnp.float32)
    zero16 = jnp.zeros((LANES,), dtype=jnp.int32)

    def init_body(k, _):
      for j in range(NVJ):
        acc[k, pl.ds(j * LANES, LANES)] = neg
      return 0
    lax.fori_loop(jnp.int32(0), jnp.int32(NUM_GRAPHS), init_body, 0)

    for q in range(NUM_GRAPHS // LANES):
      bstart[pl.ds(q * LANES, LANES)] = zero16
      bend[pl.ds(q * LANES, LANES)] = zero16

    ids_h.wait()
    iota = lax.iota(jnp.int32, LANES)
    sent_idx = jnp.where(iota == 0, off0 + IDS_PAD - 1,
                         jnp.int32(IDS_PAD + R_PER_W))
    sent_val = jnp.where(iota == 0, jnp.int32(-1), jnp.int32(-2))
    plsc.store_scatter(ids, [sent_idx], sent_val, mask=iota < 2)


    def scan_body(g, _):
      a0 = off0 + IDS_PAD + g * LANES
      cur = ids[pl.ds(a0, LANES)]
      prev = ids[pl.ds(a0 - 1, LANES)]
      nxt = ids[pl.ds(a0 + 1, LANES)]
      val = g * LANES + iota
      plsc.store_scatter(bstart, [cur], val, mask=cur != prev)
      plsc.store_scatter(bend, [cur], val + 1, mask=cur != nxt)
      return 0
    lax.fori_loop(jnp.int32(0), lax.div(rcount, jnp.int32(LANES)), scan_body, 0)

    def chunk_lb(c):
      return jnp.minimum(jnp.int32(c * CHUNK), rcount - CHUNK)

    def process(buf, lb):
      s_lo = sget(ids, off0 + IDS_PAD + lb)
      s_hi = sget(ids, off0 + IDS_PAD + lb + CHUNK - 1)

      def seg_body(s, _):
        lo = jnp.maximum(sget(bstart, s), lb)
        hi = jnp.minimum(sget(bend, s), lb + CHUNK)
        hi = jnp.maximum(hi, lo)
        a = tuple(acc[s, pl.ds(j * LANES, LANES)] for j in range(NVJ))

        def row_body(i, a):
          r = i - lb
          return tuple(
              jnp.maximum(a[j], buf[r, pl.ds(j * LANES, LANES)])
              for j in range(NVJ))
        a = lax.fori_loop(lo, hi, row_body, a)
        for j in range(NVJ):
          acc[s, pl.ds(j * LANES, LANES)] = a[j]
        return 0
      lax.fori_loop(s_lo, s_hi + 1, seg_body, 0)

    bufs, sems = (buf0, buf1), (sem0, sem1)
    handles = [None, None]
    handles[0] = pltpu.async_copy(
        z_hbm.at[pl.ds(rbase + chunk_lb(0), CHUNK), :], buf0, sem0)
    for c in range(NCHUNKS):
      b = c % 2
      handles[b].wait()
      if c + 1 < NCHUNKS:
        nb = (c + 1) % 2
        handles[nb] = pltpu.async_copy(
            z_hbm.at[pl.ds(rbase + chunk_lb(c + 1), CHUNK), :],
            bufs[nb], sems[nb])
      process(bufs[b], chunk_lb(c))

    pcount = jnp.minimum(jnp.int32(PATCH_PER_W), N_NODES - pbase)
    pids_h.wait()
    for a in range(NSUPER_W):
      a0 = jnp.int32(a * 64)
      sz = jnp.minimum(jnp.int32(64), pcount - a0)
      szc = jnp.maximum(sz, jnp.int32(16))
      base_l = jnp.minimum(poff + IDS_PAD + a0, jnp.int32(PIDS_LEN - 80))
      s_first = sget(pids, base_l)
      s_last = sget(pids, base_l + szc - 1)
      needs = (a0 < pcount) & ((s_first != s_last) | (sz < 64))

      @pl.when(needs)
      def _(a0=a0, base_l=base_l, sz=sz, szc=szc):
        shift = jnp.int32(64) - szc
        dstart = pl.multiple_of(pbase + a0 - shift, 8)
        pltpu.sync_copy(z_hbm.at[pl.ds(dstart, 64), :],
                        buf0.at[pl.ds(0, 64), :])

        def prow(r, carry):
          sr = sget(pids, base_l + r)
          for j in range(NVJ):
            acc[sr, pl.ds(j * LANES, LANES)] = jnp.maximum(
                acc[sr, pl.ds(j * LANES, LANES)],
                buf0[r + shift, pl.ds(j * LANES, LANES)])
          return carry
        lax.fori_loop(jnp.int32(0), szc, prow, jnp.int32(0))

    pltpu.sync_copy(acc, out_hbm.at[wid])

  return body(z, batch_i32)


K_SPLIT = SC_END
TC_ROWS_PAD = 32768
TC_NB = TC_ROWS_PAD // 2048
TC_NSUP = TC_ROWS_PAD // 64


def _tc_block_reduce(z_full, ids3):
  def body(z_ref, i_ref, v_ref, s_ref):
    for a in range(32):
      first = i_ref[0, 0, a * 64:a * 64 + 1]
      last = i_ref[0, 0, a * 64 + 63:a * 64 + 64]
      valid = first == last
      m = jnp.max(z_ref[a * 64:(a + 1) * 64, :], axis=0, keepdims=True)
      v_ref[a:a + 1, :] = jnp.where(valid[:, None], m, NEG_INF)
      s_ref[a:a + 1, :] = jnp.where(valid, first, jnp.int32(-1))[:, None]

  return pl.pallas_call(
      body,
      grid=(TC_NB,),
      in_specs=[
          pl.BlockSpec((2048, HIDDEN),
                       lambda b: (b + K_SPLIT // 2048, jnp.int32(0))),
          pl.BlockSpec((1, 1, 2048),
                       lambda b: (b, jnp.int32(0), jnp.int32(0))),
      ],
      out_specs=[
          pl.BlockSpec((32, HIDDEN), lambda b: (b, jnp.int32(0))),
          pl.BlockSpec((32, 1), lambda b: (b, jnp.int32(0))),
      ],
      out_shape=[
          jax.ShapeDtypeStruct((TC_NSUP, HIDDEN), jnp.float32),
          jax.ShapeDtypeStruct((TC_NSUP, 1), jnp.int32),
      ],
  )(z_full, ids3)


def _tc_merge_head(partials, supvals, supids2, W, b2):
  def body(p_ref, v_ref, s_ref, w_ref, b_ref, o_ref, tmp):
    tmp[...] = jnp.max(p_ref[...], axis=0)

    for g in range(TC_NSUP // 8):
      gf = s_ref[8 * g, 0]
      gl = s_ref[8 * g + 7, 0]
      guni = (gf == gl) & (gf >= 0)

      @pl.when(guni)
      def _(g=g, gf=gf):
        m = jnp.max(v_ref[8 * g:8 * g + 8, :], axis=0, keepdims=True)
        tmp[pl.ds(gf, 1), :] = jnp.maximum(tmp[pl.ds(gf, 1), :], m)

      @pl.when(jnp.logical_not(guni))
      def _(g=g):
        for r in range(8 * g, 8 * g + 8):
          sr = s_ref[r, 0]

          @pl.when(sr >= 0)
          def _(sr=sr, r=r):
            tmp[pl.ds(sr, 1), :] = jnp.maximum(
                tmp[pl.ds(sr, 1), :], v_ref[r:r + 1, :])
    o_ref[...] = lax.dot_general(
        tmp[...], w_ref[...], (((1,), (1,)), ((), ())),
        preferred_element_type=jnp.float32) + b_ref[...]

  return pl.pallas_call(
      body,
      in_specs=[
          pl.BlockSpec(memory_space=pltpu.ANY) if False else pl.BlockSpec(),
          pl.BlockSpec(),
          pl.BlockSpec(memory_space=pltpu.SMEM),
          pl.BlockSpec(),
          pl.BlockSpec(),
      ],
      scratch_shapes=[pltpu.VMEM((NUM_GRAPHS, HIDDEN), jnp.float32)],
      out_shape=jax.ShapeDtypeStruct((NUM_GRAPHS, W.shape[0]), jnp.float32),
  )(partials, supvals, supids2, W, b2)


def kernel(z, edge_index, batch, W, b):
  del edge_index
  batch_i32 = batch.astype(jnp.int32)
  partials = _sc_segment_max(z, batch_i32)
  ids_pad = jnp.concatenate(
      [batch_i32[K_SPLIT:],
       jnp.full((TC_ROWS_PAD - (N_NODES - K_SPLIT),), -1, jnp.int32)])
  supvals, supids2 = _tc_block_reduce(z, ids_pad.reshape(TC_NB, 1, 2048))
  return _tc_merge_head(partials, supvals, supids2, W,
                        b.reshape(1, -1).astype(jnp.float32))


# speedup vs baseline: 1.1035x; 1.1035x over previous
import functools

import jax
import jax.numpy as jnp
from jax import lax
from jax.experimental import pallas as pl
from jax.experimental.pallas import tpu as pltpu
from jax.experimental.pallas import tpu_sc as plsc

N_NODES = 50000
HIDDEN = 256
NUM_GRAPHS = 64
NW = 32
SC_END = 18432
R_PER_W = 576
CHUNK = 128
NCHUNKS = 5
PATCH_PER_W = 1024
NSUPER_W = 16
PIDS_LEN = 8 + PATCH_PER_W + 96
LANES = 16
NVJ = HIDDEN // LANES
IDS_PAD = 8
NEG_INF = float("-inf")


def _sc_segment_max(z, batch_i32):
  mesh = plsc.VectorSubcoreMesh(core_axis_name="c", subcore_axis_name="s")

  @functools.partial(
      pl.kernel,
      mesh=mesh,
      compiler_params=pltpu.CompilerParams(needs_layout_passes=False),
      out_type=jax.ShapeDtypeStruct((NW, NUM_GRAPHS, HIDDEN), jnp.float32),
      scratch_types=[
          pltpu.VMEM((CHUNK, HIDDEN), jnp.float32),
          pltpu.VMEM((CHUNK, HIDDEN), jnp.float32),
          pltpu.VMEM((IDS_PAD + R_PER_W + 24,), jnp.int32),
          pltpu.VMEM((NUM_GRAPHS, HIDDEN), jnp.float32),
          pltpu.VMEM((NUM_GRAPHS + LANES,), jnp.int32),
          pltpu.VMEM((NUM_GRAPHS + LANES,), jnp.int32),
          pltpu.VMEM((PIDS_LEN,), jnp.int32),
          pltpu.SemaphoreType.DMA,
          pltpu.SemaphoreType.DMA,
          pltpu.SemaphoreType.DMA,
          pltpu.SemaphoreType.DMA,
      ],
  )
  def body(z_hbm, batch_hbm, out_hbm, buf0, buf1, ids, acc, bstart, bend,
           pids, sem0, sem1, sem_ids, sem_pids):
    def sget(ref, idx):
      return ref[pl.ds(idx, LANES)][0]

    wid = lax.axis_index("c") * jnp.int32(16) + lax.axis_index("s")
    rbase = wid * R_PER_W
    rcount = jnp.minimum(R_PER_W, SC_END - rbase)
    ids_base = jnp.minimum(rbase, SC_END - R_PER_W)
    off0 = rbase - ids_base

    ids_h = pltpu.async_copy(batch_hbm.at[pl.ds(ids_base, R_PER_W)],
                             ids.at[pl.ds(IDS_PAD, R_PER_W)], sem_ids)
    pbase = SC_END + wid * PATCH_PER_W
    pids_base = jnp.minimum(pbase, N_NODES - PATCH_PER_W)
    poff = pbase - pids_base
    pids_h = pltpu.async_copy(batch_hbm.at[pl.ds(pids_base, PATCH_PER_W)],
                              pids.at[pl.ds(IDS_PAD, PATCH_PER_W)], sem_pids)
    neg = jnp.full((LANES,), NEG_INF, dtype=jnp.float32)
    zero16 = jnp.zeros((LANES,), dtype=jnp.int32)

    def init_body(k, _):
      for j in range(NVJ):
        acc[k, pl.ds(j * LANES, LANES)] = neg
      return 0
    lax.fori_loop(jnp.int32(0), jnp.int32(NUM_GRAPHS), init_body, 0)

    for q in range(NUM_GRAPHS // LANES):
      bstart[pl.ds(q * LANES, LANES)] = zero16
      bend[pl.ds(q * LANES, LANES)] = zero16

    ids_h.wait()
    iota = lax.iota(jnp.int32, LANES)
    sent_idx = jnp.where(iota == 0, off0 + IDS_PAD - 1,
                         jnp.int32(IDS_PAD + R_PER_W))
    sent_val = jnp.where(iota == 0, jnp.int32(-1), jnp.int32(-2))
    plsc.store_scatter(ids, [sent_idx], sent_val, mask=iota < 2)


    def scan_body(g, _):
      a0 = off0 + IDS_PAD + g * LANES
      cur = ids[pl.ds(a0, LANES)]
      prev = ids[pl.ds(a0 - 1, LANES)]
      nxt = ids[pl.ds(a0 + 1, LANES)]
      val = g * LANES + iota
      plsc.store_scatter(bstart, [cur], val, mask=cur != prev)
      plsc.store_scatter(bend, [cur], val + 1, mask=cur != nxt)
      return 0
    lax.fori_loop(jnp.int32(0), lax.div(rcount, jnp.int32(LANES)), scan_body, 0)

    def chunk_lb(c):
      return jnp.minimum(jnp.int32(c * CHUNK), rcount - CHUNK)

    def process(buf, lb):
      s_lo = sget(ids, off0 + IDS_PAD + lb)
      s_hi = sget(ids, off0 + IDS_PAD + lb + CHUNK - 1)

      def seg_body(s, _):
        lo = jnp.maximum(sget(bstart, s), lb)
        hi = jnp.minimum(sget(bend, s), lb + CHUNK)
        hi = jnp.maximum(hi, lo)
        a = tuple(acc[s, pl.ds(j * LANES, LANES)] for j in range(NVJ))

        def row_body(i, a):
          r = i - lb
          return tuple(
              jnp.maximum(a[j], buf[r, pl.ds(j * LANES, LANES)])
              for j in range(NVJ))
        a = lax.fori_loop(lo, hi, row_body, a)
        for j in range(NVJ):
          acc[s, pl.ds(j * LANES, LANES)] = a[j]
        return 0
      lax.fori_loop(s_lo, s_hi + 1, seg_body, 0)

    bufs, sems = (buf0, buf1), (sem0, sem1)
    handles = [None, None]
    handles[0] = pltpu.async_copy(
        z_hbm.at[pl.ds(rbase + chunk_lb(0), CHUNK), :], buf0, sem0)
    for c in range(NCHUNKS):
      b = c % 2
      handles[b].wait()
      if c + 1 < NCHUNKS:
        nb = (c + 1) % 2
        handles[nb] = pltpu.async_copy(
            z_hbm.at[pl.ds(rbase + chunk_lb(c + 1), CHUNK), :],
            bufs[nb], sems[nb])
      process(bufs[b], chunk_lb(c))

    pcount = jnp.minimum(jnp.int32(PATCH_PER_W), N_NODES - pbase)
    pids_h.wait()
    for a in range(NSUPER_W):
      a0 = jnp.int32(a * 64)
      sz = jnp.minimum(jnp.int32(64), pcount - a0)
      szc = jnp.maximum(sz, jnp.int32(16))
      base_l = jnp.minimum(poff + IDS_PAD + a0, jnp.int32(PIDS_LEN - 80))
      s_first = sget(pids, base_l)
      s_last = sget(pids, base_l + szc - 1)
      needs = (a0 < pcount) & ((s_first != s_last) | (sz < 64))

      @pl.when(needs)
      def _(a0=a0, base_l=base_l, sz=sz, szc=szc):
        shift = jnp.int32(64) - szc
        dstart = pl.multiple_of(pbase + a0 - shift, 8)
        pltpu.sync_copy(z_hbm.at[pl.ds(dstart, 64), :],
                        buf0.at[pl.ds(0, 64), :])

        def prow(r, carry):
          sr = sget(pids, base_l + r)
          for j in range(NVJ):
            acc[sr, pl.ds(j * LANES, LANES)] = jnp.maximum(
                acc[sr, pl.ds(j * LANES, LANES)],
                buf0[r + shift, pl.ds(j * LANES, LANES)])
          return carry
        lax.fori_loop(jnp.int32(0), szc, prow, jnp.int32(0))

    pltpu.sync_copy(acc, out_hbm.at[wid])

  return body(z, batch_i32)


K_SPLIT = SC_END
TC_ROWS_PAD = 32768
TC_NB = TC_ROWS_PAD // 2048
TC_NSUP = TC_ROWS_PAD // 64


def _tc_block_reduce(z_full, ids3):
  def body(z_ref, i_ref, o_ref):
    b = pl.program_id(0)

    @pl.when(b == 0)
    def _():
      o_ref[...] = jnp.full((NUM_GRAPHS, HIDDEN), NEG_INF, jnp.float32)

    for a in range(32):
      first = i_ref[b, 0, a * 64]
      last = i_ref[b, 0, a * 64 + 63]
      valid = (first == last) & (first >= 0)
      m = jnp.max(z_ref[a * 64:(a + 1) * 64, :], axis=0, keepdims=True)

      @pl.when(valid)
      def _(first=first, m=m):
        o_ref[pl.ds(first, 1), :] = jnp.maximum(o_ref[pl.ds(first, 1), :], m)

  return pl.pallas_call(
      body,
      grid=(TC_NB,),
      in_specs=[
          pl.BlockSpec((2048, HIDDEN),
                       lambda b: (b + K_SPLIT // 2048, jnp.int32(0))),
          pl.BlockSpec((TC_NB, 1, 2048),
                       lambda b: (jnp.int32(0), jnp.int32(0), jnp.int32(0)),
                       memory_space=pltpu.SMEM),
      ],
      out_specs=pl.BlockSpec((NUM_GRAPHS, HIDDEN),
                             lambda b: (jnp.int32(0), jnp.int32(0))),
      out_shape=jax.ShapeDtypeStruct((NUM_GRAPHS, HIDDEN), jnp.float32),
  )(z_full, ids3)


def _tc_merge_head(partials, tc_max, W, b2):
  def body(p_ref, a_ref, w_ref, b_ref, o_ref):
    pooled = jnp.maximum(jnp.max(p_ref[...], axis=0), a_ref[...])
    o_ref[...] = lax.dot_general(
        pooled, w_ref[...], (((1,), (1,)), ((), ())),
        preferred_element_type=jnp.float32) + b_ref[...]

  return pl.pallas_call(
      body,
      out_shape=jax.ShapeDtypeStruct((NUM_GRAPHS, W.shape[0]), jnp.float32),
  )(partials, tc_max, W, b2)


def kernel(z, edge_index, batch, W, b):
  del edge_index
  batch_i32 = batch.astype(jnp.int32)
  partials = _sc_segment_max(z, batch_i32)
  ids_pad = jnp.concatenate(
      [batch_i32[K_SPLIT:],
       jnp.full((TC_ROWS_PAD - (N_NODES - K_SPLIT),), -1, jnp.int32)])
  tc_max = _tc_block_reduce(z, ids_pad.reshape(TC_NB, 1, 2048))
  return _tc_merge_head(partials, tc_max, W,
                        b.reshape(1, -1).astype(jnp.float32))


# speedup vs baseline: 1.1166x; 1.0119x over previous
import functools

import jax
import jax.numpy as jnp
from jax import lax
from jax.experimental import pallas as pl
from jax.experimental.pallas import tpu as pltpu
from jax.experimental.pallas import tpu_sc as plsc

N_NODES = 50000
HIDDEN = 256
NUM_GRAPHS = 64
NW = 32
SC_END = 18432
R_PER_W = 576
CHUNK = 128
NCHUNKS = 5
PATCH_PER_W = 1024
NSUPER_W = 32
PIDS_LEN = 8 + PATCH_PER_W + 96
LANES = 16
NVJ = HIDDEN // LANES
IDS_PAD = 8
NEG_INF = float("-inf")


def _sc_segment_max(z, batch_i32):
  mesh = plsc.VectorSubcoreMesh(core_axis_name="c", subcore_axis_name="s")

  @functools.partial(
      pl.kernel,
      mesh=mesh,
      compiler_params=pltpu.CompilerParams(needs_layout_passes=False),
      out_type=jax.ShapeDtypeStruct((NW, NUM_GRAPHS, HIDDEN), jnp.float32),
      scratch_types=[
          pltpu.VMEM((CHUNK, HIDDEN), jnp.float32),
          pltpu.VMEM((CHUNK, HIDDEN), jnp.float32),
          pltpu.VMEM((IDS_PAD + R_PER_W + 24,), jnp.int32),
          pltpu.VMEM((NUM_GRAPHS, HIDDEN), jnp.float32),
          pltpu.VMEM((NUM_GRAPHS + LANES,), jnp.int32),
          pltpu.VMEM((NUM_GRAPHS + LANES,), jnp.int32),
          pltpu.VMEM((PIDS_LEN,), jnp.int32),
          pltpu.SemaphoreType.DMA,
          pltpu.SemaphoreType.DMA,
          pltpu.SemaphoreType.DMA,
          pltpu.SemaphoreType.DMA,
      ],
  )
  def body(z_hbm, batch_hbm, out_hbm, buf0, buf1, ids, acc, bstart, bend,
           pids, sem0, sem1, sem_ids, sem_pids):
    def sget(ref, idx):
      return ref[pl.ds(idx, LANES)][0]

    wid = lax.axis_index("c") * jnp.int32(16) + lax.axis_index("s")
    rbase = wid * R_PER_W
    rcount = jnp.minimum(R_PER_W, SC_END - rbase)
    ids_base = jnp.minimum(rbase, SC_END - R_PER_W)
    off0 = rbase - ids_base

    ids_h = pltpu.async_copy(batch_hbm.at[pl.ds(ids_base, R_PER_W)],
                             ids.at[pl.ds(IDS_PAD, R_PER_W)], sem_ids)
    pbase = SC_END + wid * PATCH_PER_W
    pids_base = jnp.minimum(pbase, N_NODES - PATCH_PER_W)
    poff = pbase - pids_base
    pids_h = pltpu.async_copy(batch_hbm.at[pl.ds(pids_base, PATCH_PER_W)],
                              pids.at[pl.ds(IDS_PAD, PATCH_PER_W)], sem_pids)
    neg = jnp.full((LANES,), NEG_INF, dtype=jnp.float32)
    zero16 = jnp.zeros((LANES,), dtype=jnp.int32)

    def init_body(k, _):
      for j in range(NVJ):
        acc[k, pl.ds(j * LANES, LANES)] = neg
      return 0
    lax.fori_loop(jnp.int32(0), jnp.int32(NUM_GRAPHS), init_body, 0)

    for q in range(NUM_GRAPHS // LANES):
      bstart[pl.ds(q * LANES, LANES)] = zero16
      bend[pl.ds(q * LANES, LANES)] = zero16

    ids_h.wait()
    iota = lax.iota(jnp.int32, LANES)
    sent_idx = jnp.where(iota == 0, off0 + IDS_PAD - 1,
                         jnp.int32(IDS_PAD + R_PER_W))
    sent_val = jnp.where(iota == 0, jnp.int32(-1), jnp.int32(-2))
    plsc.store_scatter(ids, [sent_idx], sent_val, mask=iota < 2)


    def scan_body(g, _):
      a0 = off0 + IDS_PAD + g * LANES
      cur = ids[pl.ds(a0, LANES)]
      prev = ids[pl.ds(a0 - 1, LANES)]
      nxt = ids[pl.ds(a0 + 1, LANES)]
      val = g * LANES + iota
      plsc.store_scatter(bstart, [cur], val, mask=cur != prev)
      plsc.store_scatter(bend, [cur], val + 1, mask=cur != nxt)
      return 0
    lax.fori_loop(jnp.int32(0), lax.div(rcount, jnp.int32(LANES)), scan_body, 0)

    def chunk_lb(c):
      return jnp.minimum(jnp.int32(c * CHUNK), rcount - CHUNK)

    def process(buf, lb):
      s_lo = sget(ids, off0 + IDS_PAD + lb)
      s_hi = sget(ids, off0 + IDS_PAD + lb + CHUNK - 1)

      def seg_body(s, _):
        lo = jnp.maximum(sget(bstart, s), lb)
        hi = jnp.minimum(sget(bend, s), lb + CHUNK)
        hi = jnp.maximum(hi, lo)
        a = tuple(acc[s, pl.ds(j * LANES, LANES)] for j in range(NVJ))

        def row_body(i, a):
          r = i - lb
          return tuple(
              jnp.maximum(a[j], buf[r, pl.ds(j * LANES, LANES)])
              for j in range(NVJ))
        a = lax.fori_loop(lo, hi, row_body, a)
        for j in range(NVJ):
          acc[s, pl.ds(j * LANES, LANES)] = a[j]
        return 0
      lax.fori_loop(s_lo, s_hi + 1, seg_body, 0)

    bufs, sems = (buf0, buf1), (sem0, sem1)
    handles = [None, None]
    handles[0] = pltpu.async_copy(
        z_hbm.at[pl.ds(rbase + chunk_lb(0), CHUNK), :], buf0, sem0)
    for c in range(NCHUNKS):
      b = c % 2
      handles[b].wait()
      if c + 1 < NCHUNKS:
        nb = (c + 1) % 2
        handles[nb] = pltpu.async_copy(
            z_hbm.at[pl.ds(rbase + chunk_lb(c + 1), CHUNK), :],
            bufs[nb], sems[nb])
      process(bufs[b], chunk_lb(c))

    pcount = jnp.minimum(jnp.int32(PATCH_PER_W), N_NODES - pbase)
    pids_h.wait()
    for a in range(NSUPER_W):
      a0 = jnp.int32(a * 32)
      sz = jnp.minimum(jnp.int32(32), pcount - a0)
      szc = jnp.maximum(sz, jnp.int32(16))
      base_l = jnp.minimum(poff + IDS_PAD + a0, jnp.int32(PIDS_LEN - 80))
      s_first = sget(pids, base_l)
      s_last = sget(pids, base_l + szc - 1)
      needs = (a0 < pcount) & ((s_first != s_last) | (sz < 32))

      @pl.when(needs)
      def _(a0=a0, base_l=base_l, sz=sz, szc=szc):
        shift = jnp.int32(32) - szc
        dstart = pl.multiple_of(pbase + a0 - shift, 8)
        pltpu.sync_copy(z_hbm.at[pl.ds(dstart, 32), :],
                        buf0.at[pl.ds(0, 32), :])

        def prow(r, carry):
          sr = sget(pids, base_l + r)
          for j in range(NVJ):
            acc[sr, pl.ds(j * LANES, LANES)] = jnp.maximum(
                acc[sr, pl.ds(j * LANES, LANES)],
                buf0[r + shift, pl.ds(j * LANES, LANES)])
          return carry
        lax.fori_loop(jnp.int32(0), szc, prow, jnp.int32(0))

    pltpu.sync_copy(acc, out_hbm.at[wid])

  return body(z, batch_i32)


K_SPLIT = SC_END
TC_ROWS_PAD = 32768
TC_NB = TC_ROWS_PAD // 2048
TC_NSUP = TC_ROWS_PAD // 64


def _tc_block_reduce(z_full, ids3):
  def body(z_ref, i_ref, o_ref):
    b = pl.program_id(0)

    @pl.when(b == 0)
    def _():
      o_ref[...] = jnp.full((NUM_GRAPHS, HIDDEN), NEG_INF, jnp.float32)

    for a in range(64):
      first = i_ref[b, 0, a * 32]
      last = i_ref[b, 0, a * 32 + 31]
      valid = (first == last) & (first >= 0)
      m = jnp.max(z_ref[a * 32:(a + 1) * 32, :], axis=0, keepdims=True)

      @pl.when(valid)
      def _(first=first, m=m):
        o_ref[pl.ds(first, 1), :] = jnp.maximum(o_ref[pl.ds(first, 1), :], m)

  return pl.pallas_call(
      body,
      grid=(TC_NB,),
      in_specs=[
          pl.BlockSpec((2048, HIDDEN),
                       lambda b: (b + K_SPLIT // 2048, jnp.int32(0))),
          pl.BlockSpec((TC_NB, 1, 2048),
                       lambda b: (jnp.int32(0), jnp.int32(0), jnp.int32(0)),
                       memory_space=pltpu.SMEM),
      ],
      out_specs=pl.BlockSpec((NUM_GRAPHS, HIDDEN),
                             lambda b: (jnp.int32(0), jnp.int32(0))),
      out_shape=jax.ShapeDtypeStruct((NUM_GRAPHS, HIDDEN), jnp.float32),
  )(z_full, ids3)


def _tc_merge_head(partials, tc_max, W, b2):
  def body(p_ref, a_ref, w_ref, b_ref, o_ref):
    pooled = jnp.maximum(jnp.max(p_ref[...], axis=0), a_ref[...])
    o_ref[...] = lax.dot_general(
        pooled, w_ref[...], (((1,), (1,)), ((), ())),
        preferred_element_type=jnp.float32) + b_ref[...]

  return pl.pallas_call(
      body,
      out_shape=jax.ShapeDtypeStruct((NUM_GRAPHS, W.shape[0]), jnp.float32),
  )(partials, tc_max, W, b2)


def kernel(z, edge_index, batch, W, b):
  del edge_index
  batch_i32 = batch.astype(jnp.int32)
  partials = _sc_segment_max(z, batch_i32)
  ids_pad = jnp.concatenate(
      [batch_i32[K_SPLIT:],
       jnp.full((TC_ROWS_PAD - (N_NODES - K_SPLIT),), -1, jnp.int32)])
  tc_max = _tc_block_reduce(z, ids_pad.reshape(TC_NB, 1, 2048))
  return _tc_merge_head(partials, tc_max, W,
                        b.reshape(1, -1).astype(jnp.float32))


# speedup vs baseline: 1.1967x; 1.0717x over previous
import functools

import jax
import jax.numpy as jnp
from jax import lax
from jax.experimental import pallas as pl
from jax.experimental.pallas import tpu as pltpu
from jax.experimental.pallas import tpu_sc as plsc

N_NODES = 50000
HIDDEN = 256
NUM_GRAPHS = 64
NW = 32
R_PER_W = 1568
CHUNK = 128
NCHUNKS = 13
LANES = 16
NVJ = HIDDEN // LANES
IDS_PAD = 8
NEG_INF = float("-inf")


def _sc_segment_max(z, batch_i32):
  mesh = plsc.VectorSubcoreMesh(core_axis_name="c", subcore_axis_name="s")

  @functools.partial(
      pl.kernel,
      mesh=mesh,
      compiler_params=pltpu.CompilerParams(needs_layout_passes=False),
      out_type=jax.ShapeDtypeStruct((NW, NUM_GRAPHS, HIDDEN), jnp.float32),
      scratch_types=[
          pltpu.VMEM((CHUNK, HIDDEN), jnp.float32),
          pltpu.VMEM((CHUNK, HIDDEN), jnp.float32),
          pltpu.VMEM((IDS_PAD + R_PER_W + 24,), jnp.int32),
          pltpu.VMEM((NUM_GRAPHS, HIDDEN), jnp.float32),
          pltpu.VMEM((NUM_GRAPHS + LANES,), jnp.int32),
          pltpu.VMEM((NUM_GRAPHS + LANES,), jnp.int32),
          pltpu.SemaphoreType.DMA,
          pltpu.SemaphoreType.DMA,
          pltpu.SemaphoreType.DMA,
      ],
  )
  def body(z_hbm, batch_hbm, out_hbm, buf0, buf1, ids, acc, bstart, bend,
           sem0, sem1, sem_ids):
    def sget(ref, idx):
      return ref[pl.ds(idx, LANES)][0]

    wid = lax.axis_index("c") * jnp.int32(16) + lax.axis_index("s")
    rbase = wid * R_PER_W
    rcount = jnp.minimum(R_PER_W, N_NODES - rbase)
    ids_base = jnp.minimum(rbase, N_NODES - R_PER_W)
    off0 = rbase - ids_base

    ids_h = pltpu.async_copy(batch_hbm.at[pl.ds(ids_base, R_PER_W)],
                             ids.at[pl.ds(IDS_PAD, R_PER_W)], sem_ids)
    neg = jnp.full((LANES,), NEG_INF, dtype=jnp.float32)
    zero16 = jnp.zeros((LANES,), dtype=jnp.int32)

    def init_body(k, _):
      for j in range(NVJ):
        acc[k, pl.ds(j * LANES, LANES)] = neg
      return 0
    lax.fori_loop(jnp.int32(0), jnp.int32(NUM_GRAPHS), init_body, 0)

    for q in range(NUM_GRAPHS // LANES):
      bstart[pl.ds(q * LANES, LANES)] = zero16
      bend[pl.ds(q * LANES, LANES)] = zero16

    ids_h.wait()
    iota = lax.iota(jnp.int32, LANES)
    sent_idx = jnp.where(iota == 0, off0 + IDS_PAD - 1,
                         jnp.int32(IDS_PAD + R_PER_W))
    sent_val = jnp.where(iota == 0, jnp.int32(-1), jnp.int32(-2))
    plsc.store_scatter(ids, [sent_idx], sent_val, mask=iota < 2)


    def scan_body(g, _):
      a0 = off0 + IDS_PAD + g * LANES
      cur = ids[pl.ds(a0, LANES)]
      prev = ids[pl.ds(a0 - 1, LANES)]
      nxt = ids[pl.ds(a0 + 1, LANES)]
      val = g * LANES + iota
      plsc.store_scatter(bstart, [cur], val, mask=cur != prev)
      plsc.store_scatter(bend, [cur], val + 1, mask=cur != nxt)
      return 0
    lax.fori_loop(jnp.int32(0), lax.div(rcount, jnp.int32(LANES)), scan_body, 0)

    def chunk_lb(c):
      return jnp.minimum(jnp.int32(c * CHUNK), rcount - CHUNK)

    def process(buf, lb):
      s_lo = sget(ids, off0 + IDS_PAD + lb)
      s_hi = sget(ids, off0 + IDS_PAD + lb + CHUNK - 1)

      def seg_body(s, _):
        lo = jnp.maximum(sget(bstart, s), lb)
        hi = jnp.minimum(sget(bend, s), lb + CHUNK)
        hi = jnp.maximum(hi, lo)
        a = tuple(acc[s, pl.ds(j * LANES, LANES)] for j in range(NVJ))

        def row_body(i, a):
          r = i - lb
          return tuple(
              jnp.maximum(a[j], buf[r, pl.ds(j * LANES, LANES)])
              for j in range(NVJ))
        a = lax.fori_loop(lo, hi, row_body, a)
        for j in range(NVJ):
          acc[s, pl.ds(j * LANES, LANES)] = a[j]
        return 0
      lax.fori_loop(s_lo, s_hi + 1, seg_body, 0)

    bufs, sems = (buf0, buf1), (sem0, sem1)
    handles = [None, None]
    handles[0] = pltpu.async_copy(
        z_hbm.at[pl.ds(rbase + chunk_lb(0), CHUNK), :], buf0, sem0)
    for c in range(NCHUNKS):
      b = c % 2
      handles[b].wait()
      if c + 1 < NCHUNKS:
        nb = (c + 1) % 2
        handles[nb] = pltpu.async_copy(
            z_hbm.at[pl.ds(rbase + chunk_lb(c + 1), CHUNK), :],
            bufs[nb], sems[nb])
      process(bufs[b], chunk_lb(c))

    pltpu.sync_copy(acc, out_hbm.at[wid])

  return body(z, batch_i32)


def _tc_merge_head(partials, W, b2):
  def body(p_ref, w_ref, b_ref, o_ref):
    pooled = jnp.max(p_ref[...], axis=0)
    o_ref[...] = lax.dot_general(
        pooled, w_ref[...], (((1,), (1,)), ((), ())),
        preferred_element_type=jnp.float32) + b_ref[...]

  return pl.pallas_call(
      body,
      out_shape=jax.ShapeDtypeStruct((NUM_GRAPHS, W.shape[0]), jnp.float32),
  )(partials, W, b2)


def kernel(z, edge_index, batch, W, b):
  del edge_index
  batch_i32 = batch.astype(jnp.int32)
  partials = _sc_segment_max(z, batch_i32)
  return _tc_merge_head(partials, W, b.reshape(1, -1).astype(jnp.float32))
